# baseline (speedup 1.0000x reference)
import functools

import jax
import jax.numpy as jnp
from jax import lax
from jax.experimental import pallas as pl
from jax.experimental.pallas import tpu as pltpu
from jax.experimental.pallas import tpu_sc as plsc

N = 10000
D = 128
E = 320000
NC = 2
NS = 16
NWORK = NC * NS
B = 128
EPW = 10240
NWIN = EPW // B
EPAD = NWORK * EPW
NPAD = N + 112
RPT = NPAD // NS
NBUF = 2
CHUNK = 8
NCHUNK = NWIN // CHUNK
_SLAB_CHUNKS = [(o, 64) for o in range(0, 512, 64)] + [(512, 64), (576, 56)]

_PASS_CACHE = {}


def _sc_pass(with_c, WB):
    if (with_c, WB) in _PASS_CACHE:
        return _PASS_CACHE[(with_c, WB)]
    NWINB = EPW // WB
    NCHUNKB = NWINB // CHUNK
    mesh = plsc.VectorSubcoreMesh(core_axis_name="c", subcore_axis_name="s")
    out_type = [jax.ShapeDtypeStruct((NC * NPAD, D), jnp.float32)]
    scratch = [pltpu.VMEM_SHARED((NPAD, D), jnp.float32)]
    nsem = 2 * NBUF
    if with_c:
        out_type.append(jax.ShapeDtypeStruct((NC * NPAD, 16), jnp.float32))
        scratch.append(pltpu.VMEM_SHARED((NPAD, 16), jnp.float32))
        nsem = 4 * NBUF
    scratch += [pltpu.SemaphoreType.DMA] * nsem
    tile_scratch = [
        pltpu.VMEM((CHUNK, WB), jnp.int32),
        pltpu.VMEM((CHUNK, WB), jnp.int32),
        *[pltpu.VMEM((WB, D), jnp.float32) for _ in range(NBUF)],
    ]
    if with_c:
        tile_scratch += [
            pltpu.VMEM((CHUNK, WB), jnp.int32),
            pltpu.VMEM((CHUNK, WB), jnp.int32),
            *[pltpu.VMEM((WB, 16), jnp.float32) for _ in range(NBUF)],
        ]

    def _inner(refs, *tiles):
        if with_c:
            (table, gA, sA, t16, gB, sB, z128, z16,
             outS, outC, accS, accC) = refs[:12]
            sems = refs[12:]
            gi, si, bufs, gi2, si2, bufs2 = (
                tiles[0], tiles[1], tiles[2:2 + NBUF],
                tiles[2 + NBUF], tiles[3 + NBUF], tiles[4 + NBUF:])
            gsem2 = sems[2 * NBUF:3 * NBUF]
            ssem2 = sems[3 * NBUF:]
        else:
            table, gA, sA, z128, outS, accS = refs[:6]
            sems = refs[6:]
            gi, si, bufs = tiles[0], tiles[1], tiles[2:]
        gsem = sems[:NBUF]
        ssem = sems[NBUF:2 * NBUF]
        c = lax.axis_index("c")
        s = lax.axis_index("s")
        wid = s * NC + c
        for off, sz in _SLAB_CHUNKS:
            pltpu.sync_copy(z128.at[pl.ds(s * RPT + off, sz)],
                            accS.at[pl.ds(s * RPT + off, sz)])
            if with_c:
                pltpu.sync_copy(z16.at[pl.ds(s * RPT + off, sz)],
                                accC.at[pl.ds(s * RPT + off, sz)])
        plsc.subcore_barrier()

        def chunk_(kc, carry):
            pltpu.sync_copy(gA.at[wid * NCHUNKB + kc], gi)
            pltpu.sync_copy(sA.at[wid * NCHUNKB + kc], si)
            if with_c:
                pltpu.sync_copy(gB.at[wid * NCHUNKB + kc], gi2)
                pltpu.sync_copy(sB.at[wid * NCHUNKB + kc], si2)

            def round_(k, carry2):
                for j in range(NBUF):
                    w = k * NBUF + j
                    pltpu.async_copy(table.at[gi.at[w]], bufs[j], gsem[j])
                    if with_c:
                        pltpu.async_copy(t16.at[gi2.at[w]], bufs2[j], gsem2[j])
                for j in range(NBUF):
                    w = k * NBUF + j
                    pltpu.make_async_copy(z128.at[pl.ds(0, WB)], bufs[j],
                                          gsem[j]).wait()
                    pltpu.async_copy(bufs[j], accS.at[si.at[w]], ssem[j],
                                     add=True)
                    if with_c:
                        pltpu.make_async_copy(z16.at[pl.ds(0, WB)], bufs2[j],
                                              gsem2[j]).wait()
                        pltpu.async_copy(bufs2[j], accC.at[si2.at[w]],
                                         ssem2[j], add=True)
                for j in range(NBUF):
                    w = k * NBUF + j
                    pltpu.make_async_copy(bufs[j], accS.at[si.at[w]],
                                          ssem[j]).wait()
                    if with_c:
                        pltpu.make_async_copy(bufs2[j], accC.at[si2.at[w]],
                                              ssem2[j]).wait()
                return carry2

            lax.fori_loop(0, CHUNK // NBUF, round_, 0)
            return carry

        lax.fori_loop(0, NCHUNKB, chunk_, 0)
        plsc.subcore_barrier()
        for off, sz in _SLAB_CHUNKS:
            pltpu.sync_copy(accS.at[pl.ds(s * RPT + off, sz)],
                            outS.at[pl.ds(c * NPAD + s * RPT + off, sz)])
            if with_c:
                pltpu.sync_copy(accC.at[pl.ds(s * RPT + off, sz)],
                                outC.at[pl.ds(c * NPAD + s * RPT + off, sz)])

    @functools.partial(
        pl.kernel,
        out_type=out_type if with_c else out_type[0],
        mesh=mesh,
        compiler_params=pltpu.CompilerParams(use_tc_tiling_on_sc=False),
        scratch_types=scratch,
    )
    def body(*refs):
        pl.run_scoped(functools.partial(_inner, refs), *tile_scratch)

    _PASS_CACHE[(with_c, WB)] = body
    return body


_HIST_CACHE = []
_HNB = 8


def _histones(sidx, zeros):
    if _HIST_CACHE:
        return _HIST_CACHE[0](sidx, zeros)
    mesh = plsc.VectorSubcoreMesh(core_axis_name="c", subcore_axis_name="s")

    def _inner(s_ref, z_ref, out_ref, acc, sems, si, buf):
        c = lax.axis_index("c")
        s = lax.axis_index("s")
        wid = s * NC + c
        ones = jnp.ones((16,), jnp.float32)

        def fill(i, carry):
            buf[i, pl.ds(0, 16)] = ones
            return carry

        lax.fori_loop(0, B, fill, 0)
        for off, sz in _SLAB_CHUNKS:
            pltpu.sync_copy(z_ref.at[pl.ds(s * RPT + off, sz)],
                            acc.at[pl.ds(s * RPT + off, sz)])
        plsc.subcore_barrier()

        def chunk_(kc, carry):
            pltpu.sync_copy(s_ref.at[wid * NCHUNK + kc], si)

            def round_(k, carry2):
                for j in range(_HNB):
                    w = k * _HNB + j
                    pltpu.async_copy(buf, acc.at[si.at[w]], sems[j], add=True)
                for j in range(_HNB):
                    w = k * _HNB + j
                    pltpu.make_async_copy(buf, acc.at[si.at[w]], sems[j]).wait()
                return carry2

            lax.fori_loop(0, CHUNK // _HNB, round_, 0)
            return carry

        lax.fori_loop(0, NCHUNK, chunk_, 0)
        plsc.subcore_barrier()
        for off, sz in _SLAB_CHUNKS:
            pltpu.sync_copy(acc.at[pl.ds(s * RPT + off, sz)],
                            out_ref.at[pl.ds(c * NPAD + s * RPT + off, sz)])

    @functools.partial(
        pl.kernel,
        out_type=jax.ShapeDtypeStruct((NC * NPAD, 16), jnp.float32),
        mesh=mesh,
        compiler_params=pltpu.CompilerParams(use_tc_tiling_on_sc=False),
        scratch_types=[
            pltpu.VMEM_SHARED((NPAD, 16), jnp.float32),
            *[pltpu.SemaphoreType.DMA for _ in range(_HNB)],
        ],
    )
    def body(s_ref, z_ref, out_ref, acc, *sems):
        pl.run_scoped(
            functools.partial(_inner, s_ref, z_ref, out_ref, acc, sems),
            pltpu.VMEM((CHUNK, B), jnp.int32),
            pltpu.VMEM((B, 16), jnp.float32),
        )

    _HIST_CACHE.append(body)
    return body(sidx, zeros)


BR = 1000


def _tc1_body(x_ref, w_ref, d_ref, g_ref, dinv_ref):
    deg = 1.0 + d_ref[0] + d_ref[1]
    dinv = lax.rsqrt(deg)
    dinv_ref[...] = dinv
    g_ref[...] = dinv[:, :1] * jnp.dot(
        x_ref[...], w_ref[...], preferred_element_type=jnp.float32)


def _tc1(x, w1, degp):
    return pl.pallas_call(
        _tc1_body,
        grid=(N // BR,),
        in_specs=[
            pl.BlockSpec((BR, D), lambda i: (i, 0)),
            pl.BlockSpec((D, D), lambda i: (0, 0)),
            pl.BlockSpec((NC, BR, 16), lambda i: (0, i, 0)),
        ],
        out_specs=[
            pl.BlockSpec((BR, D), lambda i: (i, 0)),
            pl.BlockSpec((BR, 16), lambda i: (i, 0)),
        ],
        out_shape=[
            jax.ShapeDtypeStruct((N, D), jnp.float32),
            jax.ShapeDtypeStruct((N, 16), jnp.float32),
        ],
    )(x, w1, degp)


def _tc2_body(s_ref, g_ref, dv_ref, b_ref, w_ref, out_ref):
    dcol = dv_ref[:, :1]
    h = jnp.maximum(dcol * (s_ref[0] + s_ref[1] + g_ref[...]) + b_ref[...], 0.0)
    out_ref[...] = dcol * jnp.dot(h, w_ref[...], preferred_element_type=jnp.float32)


def _tc2(sp, g1, dinv16, b1, w2):
    return pl.pallas_call(
        _tc2_body,
        grid=(N // BR,),
        in_specs=[
            pl.BlockSpec((NC, BR, D), lambda i: (0, i, 0)),
            pl.BlockSpec((BR, D), lambda i: (i, 0)),
            pl.BlockSpec((BR, 16), lambda i: (i, 0)),
            pl.BlockSpec((1, D), lambda i: (0, 0)),
            pl.BlockSpec((D, D), lambda i: (0, 0)),
        ],
        out_specs=pl.BlockSpec((BR, D), lambda i: (i, 0)),
        out_shape=jax.ShapeDtypeStruct((N, D), jnp.float32),
    )(sp, g1, dinv16, b1, w2)


def _tc3_body(s_ref, g_ref, dv_ref, c_ref, b2_ref, w3_ref, b3_ref,
              out_ref, acc_ref):
    i = pl.program_id(0)

    @pl.when(i == 0)
    def _():
        acc_ref[...] = jnp.zeros_like(acc_ref)

    dv = dv_ref[...]
    dcol = dv[:, :1]
    h2 = jnp.maximum(dcol * (s_ref[0] + s_ref[1] + g_ref[...]) + b2_ref[...], 0.0)
    w16 = dv * (dv + c_ref[0] + c_ref[1])
    acc_ref[...] += jnp.sum(w16[:, :1] * h2, axis=0, keepdims=True)

    @pl.when(i == pl.num_programs(0) - 1)
    def _():
        out_ref[...] = jnp.dot(
            acc_ref[...], w3_ref[...], preferred_element_type=jnp.float32
        ) * (1.0 / N) + b3_ref[...]


def _tc3(sp, g2, dinv16, cp, b2, w3, b3):
    return pl.pallas_call(
        _tc3_body,
        grid=(N // BR,),
        in_specs=[
            pl.BlockSpec((NC, BR, D), lambda i: (0, i, 0)),
            pl.BlockSpec((BR, D), lambda i: (i, 0)),
            pl.BlockSpec((BR, 16), lambda i: (i, 0)),
            pl.BlockSpec((NC, BR, 16), lambda i: (0, i, 0)),
            pl.BlockSpec((1, D), lambda i: (0, 0)),
            pl.BlockSpec((D, D), lambda i: (0, 0)),
            pl.BlockSpec((1, D), lambda i: (0, 0)),
        ],
        out_specs=pl.BlockSpec((1, D), lambda i: (0, 0)),
        out_shape=jax.ShapeDtypeStruct((1, D), jnp.float32),
        scratch_shapes=[pltpu.VMEM((1, D), jnp.float32)],
    )(sp, g2, dinv16, cp, b2, w3, b3)


def kernel(x, edge_index, W1, b1, W2, b2, W3, b3):
    ei = edge_index.astype(jnp.int32)
    src, dst = ei[0], ei[1]
    pad = EPAD - E
    padidx = jnp.arange(pad, dtype=jnp.int32)
    zpad = (padidx * 37) % N
    trash = N + padidx % 112
    def _shape(a, wb):
        return a.reshape(NWORK * (EPW // wb) // CHUNK, CHUNK, wb)

    src_gf = jnp.concatenate([src, zpad])
    dst_sf = jnp.concatenate([dst, trash])
    dst_gf = jnp.concatenate([dst, zpad])
    src_sf = jnp.concatenate([src, trash])
    src_g, dst_s = _shape(src_gf, B), _shape(dst_sf, B)

    zeros128 = jnp.zeros((NPAD, D), jnp.float32)
    zeros16 = jnp.zeros((NPAD, 16), jnp.float32)

    degp = _histones(dst_s, zeros16).reshape(NC, NPAD, 16)
    g1, dinv16 = _tc1(x, W1, degp)

    WBM = 64
    s1p, cp = _sc_pass(True, WBM)(g1, _shape(src_gf, WBM), _shape(dst_sf, WBM),
                                  dinv16, _shape(dst_gf, WBM),
                                  _shape(src_sf, WBM), zeros128, zeros16)
    g2 = _tc2(s1p.reshape(NC, NPAD, D), g1, dinv16, b1.reshape(1, D), W2)

    s2p = _sc_pass(False, B)(g2, src_g, dst_s, zeros128)
    out = _tc3(s2p.reshape(NC, NPAD, D), g2, dinv16,
               cp.reshape(NC, NPAD, 16),
               b2.reshape(1, D), W3, b3.reshape(1, D))
    return out[0]


# speedup vs baseline: 1.0873x; 1.0873x over previous
import functools

import jax
import jax.numpy as jnp
from jax import lax
from jax.experimental import pallas as pl
from jax.experimental.pallas import tpu as pltpu
from jax.experimental.pallas import tpu_sc as plsc

N = 10000
D = 128
E = 320000
NC = 2
NS = 16
NWORK = NC * NS
B = 128
EPW = 10240
NWIN = EPW // B
EPAD = NWORK * EPW
NPAD = N + 112
RPT = NPAD // NS
NBUF = 2
CHUNK = 8
NCHUNK = NWIN // CHUNK
_SLAB_CHUNKS = [(o, 64) for o in range(0, 512, 64)] + [(512, 64), (576, 56)]

_PASS_CACHE = {}


def _sc_pass(with_c, WB):
    if (with_c, WB) in _PASS_CACHE:
        return _PASS_CACHE[(with_c, WB)]
    NWINB = EPW // WB
    NCHUNKB = NWINB // CHUNK
    mesh = plsc.VectorSubcoreMesh(core_axis_name="c", subcore_axis_name="s")
    out_type = [jax.ShapeDtypeStruct((NC * NPAD, D), jnp.float32)]
    scratch = [pltpu.VMEM_SHARED((NPAD, D), jnp.float32)]
    nsem = 2 * NBUF
    if with_c:
        out_type.append(jax.ShapeDtypeStruct((NC * NPAD, 16), jnp.float32))
        scratch.append(pltpu.VMEM_SHARED((NPAD, 16), jnp.float32))
        nsem = 4 * NBUF
    scratch += [pltpu.SemaphoreType.DMA] * nsem
    tile_scratch = [
        pltpu.VMEM((CHUNK, WB), jnp.int32),
        pltpu.VMEM((CHUNK, WB), jnp.int32),
        *[pltpu.VMEM((WB, D), jnp.float32) for _ in range(NBUF)],
    ]
    if with_c:
        tile_scratch += [
            pltpu.VMEM((CHUNK, WB), jnp.int32),
            *[pltpu.VMEM((WB, 16), jnp.float32) for _ in range(NBUF)],
        ]

    def _inner(refs, *tiles):
        if with_c:
            (table, gA, sA, t16, sB, z128, z16,
             outS, outC, accS, accC) = refs[:11]
            sems = refs[11:]
            gi, si, bufs, si2, bufs2 = (
                tiles[0], tiles[1], tiles[2:2 + NBUF],
                tiles[2 + NBUF], tiles[3 + NBUF:])
            gsem2 = sems[2 * NBUF:3 * NBUF]
            ssem2 = sems[3 * NBUF:]
        else:
            table, gA, sA, z128, outS, accS = refs[:6]
            sems = refs[6:]
            gi, si, bufs = tiles[0], tiles[1], tiles[2:]
        gsem = sems[:NBUF]
        ssem = sems[NBUF:2 * NBUF]
        c = lax.axis_index("c")
        s = lax.axis_index("s")
        wid = s * NC + c
        for off, sz in _SLAB_CHUNKS:
            pltpu.sync_copy(z128.at[pl.ds(s * RPT + off, sz)],
                            accS.at[pl.ds(s * RPT + off, sz)])
            if with_c:
                pltpu.sync_copy(z16.at[pl.ds(s * RPT + off, sz)],
                                accC.at[pl.ds(s * RPT + off, sz)])
        plsc.subcore_barrier()

        def chunk_(kc, carry):
            pltpu.sync_copy(gA.at[wid * NCHUNKB + kc], gi)
            pltpu.sync_copy(sA.at[wid * NCHUNKB + kc], si)
            if with_c:
                pltpu.sync_copy(sB.at[wid * NCHUNKB + kc], si2)

            def round_(k, carry2):
                for j in range(NBUF):
                    w = k * NBUF + j
                    pltpu.async_copy(table.at[gi.at[w]], bufs[j], gsem[j])
                    if with_c:
                        pltpu.async_copy(t16.at[si.at[w]], bufs2[j], gsem2[j])
                for j in range(NBUF):
                    w = k * NBUF + j
                    pltpu.make_async_copy(z128.at[pl.ds(0, WB)], bufs[j],
                                          gsem[j]).wait()
                    pltpu.async_copy(bufs[j], accS.at[si.at[w]], ssem[j],
                                     add=True)
                    if with_c:
                        pltpu.make_async_copy(z16.at[pl.ds(0, WB)], bufs2[j],
                                              gsem2[j]).wait()
                        pltpu.async_copy(bufs2[j], accC.at[si2.at[w]],
                                         ssem2[j], add=True)
                for j in range(NBUF):
                    w = k * NBUF + j
                    pltpu.make_async_copy(bufs[j], accS.at[si.at[w]],
                                          ssem[j]).wait()
                    if with_c:
                        pltpu.make_async_copy(bufs2[j], accC.at[si2.at[w]],
                                              ssem2[j]).wait()
                return carry2

            lax.fori_loop(0, CHUNK // NBUF, round_, 0)
            return carry

        lax.fori_loop(0, NCHUNKB, chunk_, 0)
        plsc.subcore_barrier()
        for off, sz in _SLAB_CHUNKS:
            pltpu.sync_copy(accS.at[pl.ds(s * RPT + off, sz)],
                            outS.at[pl.ds(c * NPAD + s * RPT + off, sz)])
            if with_c:
                pltpu.sync_copy(accC.at[pl.ds(s * RPT + off, sz)],
                                outC.at[pl.ds(c * NPAD + s * RPT + off, sz)])

    @functools.partial(
        pl.kernel,
        out_type=out_type if with_c else out_type[0],
        mesh=mesh,
        compiler_params=pltpu.CompilerParams(use_tc_tiling_on_sc=False),
        scratch_types=scratch,
    )
    def body(*refs):
        pl.run_scoped(functools.partial(_inner, refs), *tile_scratch)

    _PASS_CACHE[(with_c, WB)] = body
    return body


_HIST_CACHE = []
_HNB = 8


def _histones(sidx, zeros):
    if _HIST_CACHE:
        return _HIST_CACHE[0](sidx, zeros)
    mesh = plsc.VectorSubcoreMesh(core_axis_name="c", subcore_axis_name="s")

    def _inner(s_ref, z_ref, out_ref, acc, sems, si, buf):
        c = lax.axis_index("c")
        s = lax.axis_index("s")
        wid = s * NC + c
        ones = jnp.ones((16,), jnp.float32)

        def fill(i, carry):
            buf[i, pl.ds(0, 16)] = ones
            return carry

        lax.fori_loop(0, B, fill, 0)
        for off, sz in _SLAB_CHUNKS:
            pltpu.sync_copy(z_ref.at[pl.ds(s * RPT + off, sz)],
                            acc.at[pl.ds(s * RPT + off, sz)])
        plsc.subcore_barrier()

        def chunk_(kc, carry):
            pltpu.sync_copy(s_ref.at[wid * NCHUNK + kc], si)

            def round_(k, carry2):
                for j in range(_HNB):
                    w = k * _HNB + j
                    pltpu.async_copy(buf, acc.at[si.at[w]], sems[j], add=True)
                for j in range(_HNB):
                    w = k * _HNB + j
                    pltpu.make_async_copy(buf, acc.at[si.at[w]], sems[j]).wait()
                return carry2

            lax.fori_loop(0, CHUNK // _HNB, round_, 0)
            return carry

        lax.fori_loop(0, NCHUNK, chunk_, 0)
        plsc.subcore_barrier()
        for off, sz in _SLAB_CHUNKS:
            pltpu.sync_copy(acc.at[pl.ds(s * RPT + off, sz)],
                            out_ref.at[pl.ds(c * NPAD + s * RPT + off, sz)])

    @functools.partial(
        pl.kernel,
        out_type=jax.ShapeDtypeStruct((NC * NPAD, 16), jnp.float32),
        mesh=mesh,
        compiler_params=pltpu.CompilerParams(use_tc_tiling_on_sc=False),
        scratch_types=[
            pltpu.VMEM_SHARED((NPAD, 16), jnp.float32),
            *[pltpu.SemaphoreType.DMA for _ in range(_HNB)],
        ],
    )
    def body(s_ref, z_ref, out_ref, acc, *sems):
        pl.run_scoped(
            functools.partial(_inner, s_ref, z_ref, out_ref, acc, sems),
            pltpu.VMEM((CHUNK, B), jnp.int32),
            pltpu.VMEM((B, 16), jnp.float32),
        )

    _HIST_CACHE.append(body)
    return body(sidx, zeros)


BR = 1000


def _tc1_body(x_ref, w_ref, d_ref, g_ref, dinv_ref):
    deg = 1.0 + d_ref[0] + d_ref[1]
    dinv = lax.rsqrt(deg)
    dinv_ref[...] = dinv
    g_ref[...] = dinv[:, :1] * jnp.dot(
        x_ref[...], w_ref[...], preferred_element_type=jnp.float32)


def _tc1(x, w1, degp):
    return pl.pallas_call(
        _tc1_body,
        grid=(N // BR,),
        in_specs=[
            pl.BlockSpec((BR, D), lambda i: (i, 0)),
            pl.BlockSpec((D, D), lambda i: (0, 0)),
            pl.BlockSpec((NC, BR, 16), lambda i: (0, i, 0)),
        ],
        out_specs=[
            pl.BlockSpec((BR, D), lambda i: (i, 0)),
            pl.BlockSpec((BR, 16), lambda i: (i, 0)),
        ],
        out_shape=[
            jax.ShapeDtypeStruct((N, D), jnp.float32),
            jax.ShapeDtypeStruct((N, 16), jnp.float32),
        ],
    )(x, w1, degp)


def _tc2_body(s_ref, g_ref, dv_ref, b_ref, w_ref, out_ref):
    dcol = dv_ref[:, :1]
    h = jnp.maximum(dcol * (s_ref[0] + s_ref[1] + g_ref[...]) + b_ref[...], 0.0)
    out_ref[...] = dcol * jnp.dot(h, w_ref[...], preferred_element_type=jnp.float32)


def _tc2(sp, g1, dinv16, b1, w2):
    return pl.pallas_call(
        _tc2_body,
        grid=(N // BR,),
        in_specs=[
            pl.BlockSpec((NC, BR, D), lambda i: (0, i, 0)),
            pl.BlockSpec((BR, D), lambda i: (i, 0)),
            pl.BlockSpec((BR, 16), lambda i: (i, 0)),
            pl.BlockSpec((1, D), lambda i: (0, 0)),
            pl.BlockSpec((D, D), lambda i: (0, 0)),
        ],
        out_specs=pl.BlockSpec((BR, D), lambda i: (i, 0)),
        out_shape=jax.ShapeDtypeStruct((N, D), jnp.float32),
    )(sp, g1, dinv16, b1, w2)


def _tc3_body(s_ref, g_ref, dv_ref, c_ref, b2_ref, w3_ref, b3_ref,
              out_ref, acc_ref):
    i = pl.program_id(0)

    @pl.when(i == 0)
    def _():
        acc_ref[...] = jnp.zeros_like(acc_ref)

    dv = dv_ref[...]
    dcol = dv[:, :1]
    h2 = jnp.maximum(dcol * (s_ref[0] + s_ref[1] + g_ref[...]) + b2_ref[...], 0.0)
    w16 = dv * (dv + c_ref[0] + c_ref[1])
    acc_ref[...] += jnp.sum(w16[:, :1] * h2, axis=0, keepdims=True)

    @pl.when(i == pl.num_programs(0) - 1)
    def _():
        out_ref[...] = jnp.dot(
            acc_ref[...], w3_ref[...], preferred_element_type=jnp.float32
        ) * (1.0 / N) + b3_ref[...]


def _tc3(sp, g2, dinv16, cp, b2, w3, b3):
    return pl.pallas_call(
        _tc3_body,
        grid=(N // BR,),
        in_specs=[
            pl.BlockSpec((NC, BR, D), lambda i: (0, i, 0)),
            pl.BlockSpec((BR, D), lambda i: (i, 0)),
            pl.BlockSpec((BR, 16), lambda i: (i, 0)),
            pl.BlockSpec((NC, BR, 16), lambda i: (0, i, 0)),
            pl.BlockSpec((1, D), lambda i: (0, 0)),
            pl.BlockSpec((D, D), lambda i: (0, 0)),
            pl.BlockSpec((1, D), lambda i: (0, 0)),
        ],
        out_specs=pl.BlockSpec((1, D), lambda i: (0, 0)),
        out_shape=jax.ShapeDtypeStruct((1, D), jnp.float32),
        scratch_shapes=[pltpu.VMEM((1, D), jnp.float32)],
    )(sp, g2, dinv16, cp, b2, w3, b3)


def kernel(x, edge_index, W1, b1, W2, b2, W3, b3):
    ei = edge_index.astype(jnp.int32)
    src, dst = ei[0], ei[1]
    pad = EPAD - E
    padidx = jnp.arange(pad, dtype=jnp.int32)
    zpad = (padidx * 37) % N
    trash = N + padidx % 112
    def _shape(a, wb):
        return a.reshape(NWORK * (EPW // wb) // CHUNK, CHUNK, wb)

    src_gf = jnp.concatenate([src, zpad])
    dst_sf = jnp.concatenate([dst, trash])
    dst_gf = jnp.concatenate([dst, zpad])
    src_sf = jnp.concatenate([src, trash])
    src_g, dst_s = _shape(src_gf, B), _shape(dst_sf, B)

    zeros128 = jnp.zeros((NPAD, D), jnp.float32)
    zeros16 = jnp.zeros((NPAD, 16), jnp.float32)

    degp = _histones(dst_s, zeros16).reshape(NC, NPAD, 16)
    g1, dinv16 = _tc1(x, W1, degp)

    dinv16p = jnp.pad(dinv16, ((0, NPAD - N), (0, 0)))
    s1p, cp = _sc_pass(True, B)(g1, src_g, dst_s, dinv16p,
                                _shape(src_sf, B), zeros128, zeros16)
    g2 = _tc2(s1p.reshape(NC, NPAD, D), g1, dinv16, b1.reshape(1, D), W2)

    s2p = _sc_pass(False, B)(g2, src_g, dst_s, zeros128)
    out = _tc3(s2p.reshape(NC, NPAD, D), g2, dinv16,
               cp.reshape(NC, NPAD, 16),
               b2.reshape(1, D), W3, b3.reshape(1, D))
    return out[0]


# speedup vs baseline: 1.0883x; 1.0010x over previous
import functools

import jax
import jax.numpy as jnp
from jax import lax
from jax.experimental import pallas as pl
from jax.experimental.pallas import tpu as pltpu
from jax.experimental.pallas import tpu_sc as plsc

N = 10000
D = 128
E = 320000
NC = 2
NS = 16
NWORK = NC * NS
B = 128
EPW = 10240
NWIN = EPW // B
EPAD = NWORK * EPW
NPAD = N + 112
RPT = NPAD // NS
NBUF = 2
CHUNK = 8
NCHUNK = NWIN // CHUNK
_SLAB_CHUNKS = [(o, 64) for o in range(0, 512, 64)] + [(512, 64), (576, 56)]

_PASS_CACHE = {}


def _sc_pass(with_c, WB):
    if (with_c, WB) in _PASS_CACHE:
        return _PASS_CACHE[(with_c, WB)]
    NWINB = EPW // WB
    NCHUNKB = NWINB // CHUNK
    mesh = plsc.VectorSubcoreMesh(core_axis_name="c", subcore_axis_name="s")
    out_type = [jax.ShapeDtypeStruct((NC * NPAD, D), jnp.float32)]
    scratch = [pltpu.VMEM_SHARED((NPAD, D), jnp.float32)]
    nsem = 2 * NBUF
    if with_c:
        out_type.append(jax.ShapeDtypeStruct((NC * NPAD, 16), jnp.float32))
        scratch.append(pltpu.VMEM_SHARED((NPAD, 16), jnp.float32))
        nsem = 4 * NBUF
    scratch += [pltpu.SemaphoreType.DMA] * nsem
    tile_scratch = [
        pltpu.VMEM((CHUNK, WB), jnp.int32),
        pltpu.VMEM((CHUNK, WB), jnp.int32),
        *[pltpu.VMEM((WB, D), jnp.float32) for _ in range(NBUF)],
    ]
    if with_c:
        tile_scratch += [
            pltpu.VMEM((CHUNK, WB), jnp.int32),
            *[pltpu.VMEM((WB, 16), jnp.float32) for _ in range(NBUF)],
        ]

    def _inner(refs, *tiles):
        if with_c:
            (table, gA, sA, t16, sB, z128, z16,
             outS, outC, accS, accC) = refs[:11]
            sems = refs[11:]
            gi, si, bufs, si2, bufs2 = (
                tiles[0], tiles[1], tiles[2:2 + NBUF],
                tiles[2 + NBUF], tiles[3 + NBUF:])
            gsem2 = sems[2 * NBUF:3 * NBUF]
            ssem2 = sems[3 * NBUF:]
        else:
            table, gA, sA, z128, outS, accS = refs[:6]
            sems = refs[6:]
            gi, si, bufs = tiles[0], tiles[1], tiles[2:]
        gsem = sems[:NBUF]
        ssem = sems[NBUF:2 * NBUF]
        c = lax.axis_index("c")
        s = lax.axis_index("s")
        wid = s * NC + c
        for off, sz in _SLAB_CHUNKS:
            pltpu.sync_copy(z128.at[pl.ds(s * RPT + off, sz)],
                            accS.at[pl.ds(s * RPT + off, sz)])
            if with_c:
                pltpu.sync_copy(z16.at[pl.ds(s * RPT + off, sz)],
                                accC.at[pl.ds(s * RPT + off, sz)])
        plsc.subcore_barrier()

        def chunk_(kc, carry):
            pltpu.sync_copy(gA.at[wid * NCHUNKB + kc], gi)
            pltpu.sync_copy(sA.at[wid * NCHUNKB + kc], si)
            if with_c:
                pltpu.sync_copy(sB.at[wid * NCHUNKB + kc], si2)

            def round_(k, carry2):
                for j in range(NBUF):
                    w = k * NBUF + j
                    pltpu.async_copy(table.at[gi.at[w]], bufs[j], gsem[j])
                    if with_c:
                        pltpu.async_copy(t16.at[si.at[w]], bufs2[j], gsem2[j])
                for j in range(NBUF):
                    w = k * NBUF + j
                    pltpu.make_async_copy(z128.at[pl.ds(0, WB)], bufs[j],
                                          gsem[j]).wait()
                    pltpu.async_copy(bufs[j], accS.at[si.at[w]], ssem[j],
                                     add=True)
                    if with_c:
                        pltpu.make_async_copy(z16.at[pl.ds(0, WB)], bufs2[j],
                                              gsem2[j]).wait()
                        pltpu.async_copy(bufs2[j], accC.at[si2.at[w]],
                                         ssem2[j], add=True)
                for j in range(NBUF):
                    w = k * NBUF + j
                    pltpu.make_async_copy(bufs[j], accS.at[si.at[w]],
                                          ssem[j]).wait()
                    if with_c:
                        pltpu.make_async_copy(bufs2[j], accC.at[si2.at[w]],
                                              ssem2[j]).wait()
                return carry2

            lax.fori_loop(0, CHUNK // NBUF, round_, 0)
            return carry

        lax.fori_loop(0, NCHUNKB, chunk_, 0)
        plsc.subcore_barrier()
        for off, sz in _SLAB_CHUNKS:
            pltpu.sync_copy(accS.at[pl.ds(s * RPT + off, sz)],
                            outS.at[pl.ds(c * NPAD + s * RPT + off, sz)])
            if with_c:
                pltpu.sync_copy(accC.at[pl.ds(s * RPT + off, sz)],
                                outC.at[pl.ds(c * NPAD + s * RPT + off, sz)])

    @functools.partial(
        pl.kernel,
        out_type=out_type if with_c else out_type[0],
        mesh=mesh,
        compiler_params=pltpu.CompilerParams(use_tc_tiling_on_sc=False),
        scratch_types=scratch,
    )
    def body(*refs):
        pl.run_scoped(functools.partial(_inner, refs), *tile_scratch)

    _PASS_CACHE[(with_c, WB)] = body
    return body


_HIST_CACHE = []
_HNB = 8


def _histones(sidx, zeros):
    if _HIST_CACHE:
        return _HIST_CACHE[0](sidx, zeros)
    mesh = plsc.VectorSubcoreMesh(core_axis_name="c", subcore_axis_name="s")

    def _inner(s_ref, z_ref, out_ref, acc, sems, si, buf):
        c = lax.axis_index("c")
        s = lax.axis_index("s")
        wid = s * NC + c
        ones = jnp.ones((16,), jnp.float32)

        def fill(i, carry):
            buf[i, pl.ds(0, 16)] = ones
            return carry

        lax.fori_loop(0, B, fill, 0)
        for off, sz in _SLAB_CHUNKS:
            pltpu.sync_copy(z_ref.at[pl.ds(s * RPT + off, sz)],
                            acc.at[pl.ds(s * RPT + off, sz)])
        plsc.subcore_barrier()

        def chunk_(kc, carry):
            pltpu.sync_copy(s_ref.at[wid * NCHUNK + kc], si)

            def round_(k, carry2):
                for j in range(_HNB):
                    w = k * _HNB + j
                    pltpu.async_copy(buf, acc.at[si.at[w]], sems[j], add=True)
                for j in range(_HNB):
                    w = k * _HNB + j
                    pltpu.make_async_copy(buf, acc.at[si.at[w]], sems[j]).wait()
                return carry2

            lax.fori_loop(0, CHUNK // _HNB, round_, 0)
            return carry

        lax.fori_loop(0, NCHUNK, chunk_, 0)
        plsc.subcore_barrier()
        for off, sz in _SLAB_CHUNKS:
            pltpu.sync_copy(acc.at[pl.ds(s * RPT + off, sz)],
                            out_ref.at[pl.ds(c * NPAD + s * RPT + off, sz)])

    @functools.partial(
        pl.kernel,
        out_type=jax.ShapeDtypeStruct((NC * NPAD, 16), jnp.float32),
        mesh=mesh,
        compiler_params=pltpu.CompilerParams(use_tc_tiling_on_sc=False),
        scratch_types=[
            pltpu.VMEM_SHARED((NPAD, 16), jnp.float32),
            *[pltpu.SemaphoreType.DMA for _ in range(_HNB)],
        ],
    )
    def body(s_ref, z_ref, out_ref, acc, *sems):
        pl.run_scoped(
            functools.partial(_inner, s_ref, z_ref, out_ref, acc, sems),
            pltpu.VMEM((CHUNK, B), jnp.int32),
            pltpu.VMEM((B, 16), jnp.float32),
        )

    _HIST_CACHE.append(body)
    return body(sidx, zeros)


BR = 1000


def _tc1_body(x_ref, w_ref, d_ref, g_ref, dinv_ref):
    deg = 1.0 + d_ref[0] + d_ref[1]
    dinv = lax.rsqrt(deg)
    dinv_ref[...] = dinv
    g_ref[...] = dinv[:, :1] * jnp.dot(
        x_ref[...], w_ref[...], preferred_element_type=jnp.float32)


def _tc1(x, w1, degp):
    return pl.pallas_call(
        _tc1_body,
        grid=(N // BR,),
        in_specs=[
            pl.BlockSpec((BR, D), lambda i: (i, 0)),
            pl.BlockSpec((D, D), lambda i: (0, 0)),
            pl.BlockSpec((NC, BR, 16), lambda i: (0, i, 0)),
        ],
        out_specs=[
            pl.BlockSpec((BR, D), lambda i: (i, 0)),
            pl.BlockSpec((BR, 16), lambda i: (i, 0)),
        ],
        out_shape=[
            jax.ShapeDtypeStruct((N, D), jnp.float32),
            jax.ShapeDtypeStruct((N, 16), jnp.float32),
        ],
    )(x, w1, degp)


def _tc2_body(s_ref, g_ref, dv_ref, b_ref, w_ref, out_ref):
    dcol = dv_ref[:, :1]
    h = jnp.maximum(dcol * (s_ref[0] + s_ref[1] + g_ref[...]) + b_ref[...], 0.0)
    out_ref[...] = dcol * jnp.dot(h, w_ref[...], preferred_element_type=jnp.float32)


def _tc2(sp, g1, dinv16, b1, w2):
    return pl.pallas_call(
        _tc2_body,
        grid=(N // BR,),
        in_specs=[
            pl.BlockSpec((NC, BR, D), lambda i: (0, i, 0)),
            pl.BlockSpec((BR, D), lambda i: (i, 0)),
            pl.BlockSpec((BR, 16), lambda i: (i, 0)),
            pl.BlockSpec((1, D), lambda i: (0, 0)),
            pl.BlockSpec((D, D), lambda i: (0, 0)),
        ],
        out_specs=pl.BlockSpec((BR, D), lambda i: (i, 0)),
        out_shape=jax.ShapeDtypeStruct((N, D), jnp.float32),
    )(sp, g1, dinv16, b1, w2)


def _tc3_body(s_ref, g_ref, dv_ref, c_ref, b2_ref, w3_ref, b3_ref,
              out_ref, acc_ref):
    i = pl.program_id(0)

    @pl.when(i == 0)
    def _():
        acc_ref[...] = jnp.zeros_like(acc_ref)

    dv = dv_ref[...]
    dcol = dv[:, :1]
    h2 = jnp.maximum(dcol * (s_ref[0] + s_ref[1] + g_ref[...]) + b2_ref[...], 0.0)
    w16 = dv * (dv + c_ref[0] + c_ref[1])
    acc_ref[...] += jnp.sum(w16[:, :1] * h2, axis=0, keepdims=True)

    @pl.when(i == pl.num_programs(0) - 1)
    def _():
        out_ref[...] = jnp.dot(
            acc_ref[...], w3_ref[...], preferred_element_type=jnp.float32
        ) * (1.0 / N) + b3_ref[...]


def _tc3(sp, g2, dinv16, cp, b2, w3, b3):
    return pl.pallas_call(
        _tc3_body,
        grid=(N // BR,),
        in_specs=[
            pl.BlockSpec((NC, BR, D), lambda i: (0, i, 0)),
            pl.BlockSpec((BR, D), lambda i: (i, 0)),
            pl.BlockSpec((BR, 16), lambda i: (i, 0)),
            pl.BlockSpec((NC, BR, 16), lambda i: (0, i, 0)),
            pl.BlockSpec((1, D), lambda i: (0, 0)),
            pl.BlockSpec((D, D), lambda i: (0, 0)),
            pl.BlockSpec((1, D), lambda i: (0, 0)),
        ],
        out_specs=pl.BlockSpec((1, D), lambda i: (0, 0)),
        out_shape=jax.ShapeDtypeStruct((1, D), jnp.float32),
        scratch_shapes=[pltpu.VMEM((1, D), jnp.float32)],
    )(sp, g2, dinv16, cp, b2, w3, b3)


def kernel(x, edge_index, W1, b1, W2, b2, W3, b3):
    ei = edge_index.astype(jnp.int32)
    src, dst = ei[0], ei[1]
    pad = EPAD - E
    padidx = jnp.arange(pad, dtype=jnp.int32)
    zpad = (padidx * 37) % N
    trash = N + padidx % 112
    def _shape(a, wb):
        return a.reshape(NWORK * (EPW // wb) // CHUNK, CHUNK, wb)

    src_gf = jnp.concatenate([src, zpad])
    dst_sf = jnp.concatenate([dst, trash])
    src_sf = jnp.concatenate([src, trash])
    src_g, dst_s = _shape(src_gf, B), _shape(dst_sf, B)

    zeros128 = jnp.zeros((NPAD, D), jnp.float32)
    zeros16 = jnp.zeros((NPAD, 16), jnp.float32)

    degp = _histones(dst_s, zeros16).reshape(NC, NPAD, 16)
    g1, dinv16 = _tc1(x, W1, degp)

    dinv16p = jnp.pad(dinv16, ((0, NPAD - N), (0, 0)))
    s1p, cp = _sc_pass(True, B)(g1, src_g, dst_s, dinv16p,
                                _shape(src_sf, B), zeros128, zeros16)
    g2 = _tc2(s1p.reshape(NC, NPAD, D), g1, dinv16, b1.reshape(1, D), W2)

    s2p = _sc_pass(False, B)(g2, src_g, dst_s, zeros128)
    out = _tc3(s2p.reshape(NC, NPAD, D), g2, dinv16,
               cp.reshape(NC, NPAD, 16),
               b2.reshape(1, D), W3, b3.reshape(1, D))
    return out[0]
